# Initial kernel scaffold; baseline (speedup 1.0000x reference)
import jax
import jax.numpy as jnp
from jax.experimental import pallas as pl


def kernel(x, edge_index, Wg, bg, W1, g1, b1, W2, g2, b2, Wd, bd):
    raise NotImplementedError("write your pallas kernel here")


# speedup vs baseline: 23.2758x; 23.2758x over previous
import functools

import jax
import jax.numpy as jnp
from jax import lax
from jax.experimental import pallas as pl
from jax.experimental.pallas import tpu as pltpu
from jax.experimental.pallas import tpu_sc as plsc

N = 10000
E = 320000
D_IN = 128
H = 7
HP = 8
NP = 10240
EPS = 1e-5

NC, NS = 2, 16
NW = NC * NS
CH = 128
RW = 80
NROWS = NW * RW
E_PAD = NROWS * CH
NB = 4
GROUPS = RW // NB
RPT = NP // NS

_MESH = plsc.VectorSubcoreMesh(core_axis_name="c", subcore_axis_name="s")
_SC_PARAMS = pltpu.CompilerParams(use_tc_tiling_on_sc=False)


def _hop_body(table, srcr, dstr, zeros, out, idx_s, idx_d, rows, gsem,
              acc, tbl):
    c = lax.axis_index("c")
    s = lax.axis_index("s")
    w = c * NS + s

    pltpu.sync_copy(table.at[pl.ds(s * RPT, RPT)], tbl.at[pl.ds(s * RPT, RPT)])
    pltpu.sync_copy(zeros.at[pl.ds(s * RPT, RPT)], acc.at[pl.ds(s * RPT, RPT)])
    pltpu.sync_copy(srcr.at[pl.ds(w * RW, RW)], idx_s)
    pltpu.sync_copy(dstr.at[pl.ds(w * RW, RW)], idx_d)
    plsc.subcore_barrier()

    for b in range(NB):
        pltpu.async_copy(tbl.at[idx_s.at[b]], rows.at[b], gsem[b])

    def group(g, carry):
        base = g * NB
        for b in range(NB):
            pltpu.make_async_copy(
                tbl.at[idx_s.at[base + b]], rows.at[b], gsem[b]).wait()
            pltpu.sync_copy(rows.at[b], acc.at[idx_d.at[base + b]], add=True)
            pltpu.async_copy(tbl.at[idx_s.at[base + NB + b]], rows.at[b],
                             gsem[b])
        return carry

    lax.fori_loop(0, GROUPS - 1, group, 0)

    last = (GROUPS - 1) * NB
    for b in range(NB):
        pltpu.make_async_copy(
            tbl.at[idx_s.at[last + b]], rows.at[b], gsem[b]).wait()
        pltpu.sync_copy(rows.at[b], acc.at[idx_d.at[last + b]], add=True)

    plsc.subcore_barrier()
    pltpu.sync_copy(acc.at[pl.ds(s * RPT, RPT)],
                    out.at[c, pl.ds(s * RPT, RPT)])


_hop = pl.kernel(
    _hop_body,
    out_type=jax.ShapeDtypeStruct((NC, NP, HP), jnp.float32),
    mesh=_MESH,
    scratch_types=[
        pltpu.VMEM((RW, CH), jnp.int32),
        pltpu.VMEM((RW, CH), jnp.int32),
        pltpu.VMEM((NB, CH, HP), jnp.float32),
        [pltpu.SemaphoreType.DMA] * NB,
        pltpu.VMEM_SHARED((NP, HP), jnp.float32),
        pltpu.VMEM_SHARED((NP, HP), jnp.float32),
    ],
    compiler_params=_SC_PARAMS,
)


def _rmask():
    return lax.broadcasted_iota(jnp.int32, (NP, 1), 0) < N


def _cmask():
    return lax.broadcasted_iota(jnp.int32, (1, HP), 1) < H


def _gate_body(x, wg, bg, o):
    z = jnp.dot(x[...], wg[...], preferred_element_type=jnp.float32) + bg[...]
    h = jax.nn.sigmoid(z)
    o[...] = jnp.where(_rmask() & _cmask(), h, 0.0)


def _norm_body(dp, h0, norm_o, u0_o):
    deg = dp[0] + dp[1]
    norm = lax.rsqrt(jnp.maximum(deg, 1.0))
    norm_o[...] = norm
    u0_o[...] = h0[...] * norm


def _scale_body(vp, norm, h_o, u_o):
    h = jnp.where(_rmask(), (vp[0] + vp[1]) * norm[...], 0.0)
    h_o[...] = h
    u_o[...] = h * norm[...]


def _bn_relu(z, g, b):
    mean = jnp.sum(z, axis=0, keepdims=True) / N
    var = jnp.sum(z * z, axis=0, keepdims=True) / N - mean * mean
    zn = (z - mean) * lax.rsqrt(var + EPS) * g + b
    return jnp.maximum(zn, 0.0)


def _layer_body(vp, norm, h0, h1, W, g, b, t_o, u_o):
    h2 = jnp.where(_rmask(), (vp[0] + vp[1]) * norm[...], 0.0)
    Wm = W[...]
    z = (jnp.dot(h0[...], Wm[0:HP], preferred_element_type=jnp.float32)
         + jnp.dot(h1[...], Wm[HP:2 * HP], preferred_element_type=jnp.float32)
         + jnp.dot(h2, Wm[2 * HP:3 * HP], preferred_element_type=jnp.float32))
    t = jnp.where(_rmask() & _cmask(), _bn_relu(z, g[...], b[...]), 0.0)
    t_o[...] = t
    u_o[...] = t * norm[...]


def _final_body(vp, norm, h0, h1, W, g, b, Wd, bd, y_o):
    h2 = jnp.where(_rmask(), (vp[0] + vp[1]) * norm[...], 0.0)
    Wm = W[...]
    z = (jnp.dot(h0[...], Wm[0:HP], preferred_element_type=jnp.float32)
         + jnp.dot(h1[...], Wm[HP:2 * HP], preferred_element_type=jnp.float32)
         + jnp.dot(h2, Wm[2 * HP:3 * HP], preferred_element_type=jnp.float32))
    t = jnp.where(_rmask() & _cmask(), _bn_relu(z, g[...], b[...]), 0.0)
    m = jnp.sum(t, axis=0, keepdims=True) / N
    y_o[...] = jnp.dot(m, Wd[...], preferred_element_type=jnp.float32) + bd[...]


def _tc(body, *outs):
    return pl.pallas_call(body, out_shape=tuple(
        jax.ShapeDtypeStruct(s, jnp.float32) for s in outs))


_gate = _tc(_gate_body, (NP, HP))
_norm = _tc(_norm_body, (NP, HP), (NP, HP))
_scale = _tc(_scale_body, (NP, HP), (NP, HP))
_layer = _tc(_layer_body, (NP, HP), (NP, HP))
_final = _tc(_final_body, (1, D_IN))


def _pad_stack_w(W):
    Wp = jnp.zeros((3 * HP, HP), jnp.float32)
    for i in range(3):
        Wp = Wp.at[i * HP:i * HP + H, :H].set(W[i * H:(i + 1) * H])
    return Wp


def _pad_vec(v):
    return jnp.zeros((1, HP), jnp.float32).at[0, :H].set(v)


def kernel(x, edge_index, Wg, bg, W1, g1, b1, W2, g2, b2, Wd, bd):
    xp = jnp.zeros((NP, D_IN), jnp.float32).at[:N].set(x)
    pad = E_PAD - E
    srcr = jnp.concatenate(
        [edge_index[0], jnp.zeros((pad,), jnp.int32)]).reshape(NROWS, CH)
    dstr = jnp.concatenate(
        [edge_index[1], jnp.full((pad,), N, jnp.int32)]).reshape(NROWS, CH)
    zeros = jnp.zeros((NP, HP), jnp.float32)
    ones = jnp.ones((NP, HP), jnp.float32)

    Wgp = jnp.zeros((D_IN, HP), jnp.float32).at[:, :H].set(Wg)
    bgp = _pad_vec(bg)
    W1p, W2p = _pad_stack_w(W1), _pad_stack_w(W2)
    g1p, b1p = _pad_vec(g1), _pad_vec(b1)
    g2p, b2p = _pad_vec(g2), _pad_vec(b2)
    Wdp = jnp.zeros((HP, D_IN), jnp.float32).at[:H].set(Wd)
    bdp = bd.reshape(1, D_IN)

    (h0,) = _gate(xp, Wgp, bgp)
    dp = _hop(ones, srcr, dstr, zeros)
    norm8, u0 = _norm(dp, h0)

    v1 = _hop(u0, srcr, dstr, zeros)
    h1, u1 = _scale(v1, norm8)
    v2 = _hop(u1, srcr, dstr, zeros)
    t1, ua = _layer(v2, norm8, h0, h1, W1p, g1p, b1p)

    w1 = _hop(ua, srcr, dstr, zeros)
    h1b, u1b = _scale(w1, norm8)
    w2 = _hop(u1b, srcr, dstr, zeros)
    (y,) = _final(w2, norm8, t1, h1b, W2p, g2p, b2p, Wdp, bdp)

    return y.reshape(D_IN)
